# Initial kernel scaffold; baseline (speedup 1.0000x reference)
import jax
import jax.numpy as jnp
from jax.experimental import pallas as pl


def kernel(items_embeddings, membership):
    raise NotImplementedError("write your pallas kernel here")


# speedup vs baseline: 10.2875x; 10.2875x over previous
import functools

import jax
import jax.numpy as jnp
from jax import lax
from jax.experimental import pallas as pl
from jax.experimental.pallas import tpu as pltpu
from jax.experimental.pallas import tpu_sc as plsc

N = 4096
D = 128
TC_R = 128
K = 3


def _tc_body(a_ref, bT_ref, sqr_ref, sqc_ref, vals_ref, idx_ref):
    a = a_ref[...]
    bT = bT_ref[...]
    gram = jnp.dot(a, bT, preferred_element_type=jnp.float32)
    d2 = sqr_ref[...] + sqc_ref[...] - 2.0 * gram
    big = jnp.float32(jnp.inf)
    work = jnp.where(d2 > 0.0, d2, big)
    cols = lax.broadcasted_iota(jnp.int32, (TC_R, N), 1)
    row0 = pl.program_id(0) * TC_R
    grows = row0 + lax.broadcasted_iota(jnp.int32, (TC_R, 1), 0)
    vs, ids = [], []
    for _ in range(K):
        m = jnp.min(work, axis=1, keepdims=True)
        sel = work == m
        idx = jnp.min(jnp.where(sel, cols, N), axis=1, keepdims=True)
        vs.append(m)
        ids.append(grows * N + idx)
        work = jnp.where(cols == idx, big, work)
    d2top = jnp.concatenate(vs, axis=1)
    vals_ref[...] = 1.0 / (jnp.sqrt(d2top) + 1.0)
    idx_ref[...] = jnp.concatenate(ids, axis=1)


def _tc_top3(x, xT, sq_col, sq_row):
    return pl.pallas_call(
        _tc_body,
        grid=(N // TC_R,),
        in_specs=[
            pl.BlockSpec((TC_R, D), lambda i: (i, 0)),
            pl.BlockSpec((D, N), lambda i: (0, 0)),
            pl.BlockSpec((TC_R, 1), lambda i: (i, 0)),
            pl.BlockSpec((1, N), lambda i: (0, 0)),
        ],
        out_specs=[
            pl.BlockSpec((TC_R, K), lambda i: (i, 0)),
            pl.BlockSpec((TC_R, K), lambda i: (i, 0)),
        ],
        out_shape=[
            jax.ShapeDtypeStruct((N, K), jnp.float32),
            jax.ShapeDtypeStruct((N, K), jnp.int32),
        ],
    )(x, xT, sq_col, sq_row)


def _sc_combine(idxT, valsT, memflat):
    info = plsc.get_sparse_core_info()
    nw = info.num_cores * info.num_subcores
    rpw = N // nw
    mesh = plsc.VectorSubcoreMesh(core_axis_name="c", subcore_axis_name="s")

    @functools.partial(
        pl.kernel,
        mesh=mesh,
        out_type=jax.ShapeDtypeStruct((N,), jnp.float32),
        scratch_types=[
            pltpu.VMEM((K, rpw), jnp.int32),
            pltpu.VMEM((K, rpw), jnp.int32),
            pltpu.VMEM((K, rpw), jnp.float32),
            pltpu.VMEM((rpw,), jnp.float32),
            pltpu.SemaphoreType.DMA,
        ],
    )
    def k(idx_hbm, vals_hbm, mem_hbm, out_hbm, idx_v, mem_v, vals_v, out_v, sem):
        wid = lax.axis_index("s") * info.num_cores + lax.axis_index("c")
        base = wid * rpw
        pltpu.sync_copy(idx_hbm.at[:, pl.ds(base, rpw)], idx_v)
        pltpu.sync_copy(vals_hbm.at[:, pl.ds(base, rpw)], vals_v)
        for kk in range(K):
            pltpu.async_copy(mem_hbm.at[idx_v.at[kk]], mem_v.at[kk], sem).wait()
        nl = info.num_lanes
        for j in range(rpw // nl):
            s = pl.ds(j * nl, nl)
            acc = vals_v[0, s] * mem_v[0, s].astype(jnp.float32)
            acc = acc + vals_v[1, s] * mem_v[1, s].astype(jnp.float32)
            acc = acc + vals_v[2, s] * mem_v[2, s].astype(jnp.float32)
            out_v[s] = acc / 3.0
        pltpu.sync_copy(out_v, out_hbm.at[pl.ds(base, rpw)])

    return k(idxT, valsT, memflat)


def kernel(items_embeddings, membership):
    x = items_embeddings
    sq = jnp.sum(x * x, axis=1)
    vals, idx = _tc_top3(x, x.T, sq[:, None], sq[None, :])
    return _sc_combine(idx.T, vals.T, membership.reshape(-1))
